# Initial kernel scaffold; baseline (speedup 1.0000x reference)
import jax
import jax.numpy as jnp
from jax.experimental import pallas as pl


def kernel(topo_5xwxh):
    raise NotImplementedError("write your pallas kernel here")


# speedup vs baseline: 35.1177x; 35.1177x over previous
import functools

import jax
import jax.numpy as jnp
from jax import lax
from jax.experimental import pallas as pl
from jax.experimental.pallas import tpu as pltpu
from jax.experimental.pallas import tpu_sc as plsc

T_STEPS = 30
BANDS = 5
W = 512
H = 512
ROWS = BANDS * W
NC = 2
NS = 16
L = 16
NW = NC * NS
RPW = ROWS // NW
RG = 4
NG = RPW // RG
VPG = RG * H // L


def _minmax_body(in_hbm, mm_hbm, in_v, pmm_v):
    cid = lax.axis_index("c")
    sid = lax.axis_index("s")
    wid = cid * NS + sid

    def chunk(k, carry):
        mn, mx = carry
        off = (wid * RPW + k * RG) * H
        pltpu.sync_copy(in_hbm.at[pl.ds(off, RG * H)], in_v)

        def vec(i, c):
            m0, m1 = c
            v = in_v[pl.ds(i * L, L)]
            return jnp.minimum(m0, v), jnp.maximum(m1, v)

        return lax.fori_loop(0, VPG, vec, (mn, mx))

    big = jnp.full((L,), jnp.inf, jnp.float32)
    mn_vec, mx_vec = lax.fori_loop(0, NG, chunk, (big, -big))
    pmm_v[0, :] = mn_vec
    pmm_v[1, :] = mx_vec
    pltpu.sync_copy(pmm_v, mm_hbm.at[wid])


def _encode_body(in_hbm, mm_hbm, out_hbm, in_v, blk, tid_v, pmm_v, all_v):
    cid = lax.axis_index("c")
    sid = lax.axis_index("s")
    wid = cid * NS + sid

    pltpu.sync_copy(mm_hbm, all_v)
    mn_vec = all_v[0, 0, :]
    mx_vec = all_v[0, 1, :]
    for k in range(1, NW):
        mn_vec = jnp.minimum(mn_vec, all_v[k, 0, :])
        mx_vec = jnp.maximum(mx_vec, all_v[k, 1, :])

    iota = lax.iota(jnp.int32, L)
    row0_idx = jnp.zeros((L,), jnp.int32)
    row1_idx = jnp.full((L,), 1, jnp.int32)
    for s in (8, 4, 2, 1):
        pmm_v[0, :] = mn_vec
        pmm_v[1, :] = mx_vec
        perm = iota ^ s
        mn_vec = jnp.minimum(mn_vec, plsc.load_gather(pmm_v, [row0_idx, perm]))
        mx_vec = jnp.maximum(mx_vec, plsc.load_gather(pmm_v, [row1_idx, perm]))
    mn = mn_vec
    recip = 1.0 / (mx_vec - mn_vec + 1e-8)

    zeros = jnp.zeros((L,), jnp.float32)
    ones = jnp.full((L,), 1.0, jnp.float32)

    def zero_t(t, _):
        for r in range(RG):
            for j in range(H // L):
                blk[t, r, pl.ds(j * L, L)] = zeros
        return 0

    lax.fori_loop(0, T_STEPS, zero_t, 0)

    def group(g, _):
        row0 = wid * RPW + g * RG
        pltpu.sync_copy(in_hbm.at[pl.ds(row0 * H, RG * H)], in_v)

        def enc_vec(i, _):
            x = in_v[pl.ds(i * L, L)]
            lat = (1.0 - (x - mn) * recip) * (T_STEPS - 1.0)
            t = jnp.clip(lat.astype(jnp.int32), 0, T_STEPS - 1)
            tid_v[pl.ds(i * L, L)] = t
            r_vec = jnp.full((L,), (i * L) // H, jnp.int32)
            j_vec = ((i * L) % H) + iota
            plsc.store_scatter(blk, [t, r_vec, j_vec], ones)
            return 0

        lax.fori_loop(0, VPG, enc_vec, 0)
        pltpu.sync_copy(blk, out_hbm.at[:, pl.ds(row0, RG), :])

        def restore_vec(i, _):
            t = tid_v[pl.ds(i * L, L)]
            r_vec = jnp.full((L,), (i * L) // H, jnp.int32)
            j_vec = ((i * L) % H) + iota
            plsc.store_scatter(blk, [t, r_vec, j_vec], zeros)
            return 0

        lax.fori_loop(0, VPG, restore_vec, 0)
        return 0

    lax.fori_loop(0, NG, group, 0)


@functools.cache
def _build():
    mesh = plsc.VectorSubcoreMesh(core_axis_name="c", subcore_axis_name="s")
    minmax = pl.kernel(
        _minmax_body,
        out_type=jax.ShapeDtypeStruct((NW, 2, L), jnp.float32),
        mesh=mesh,
        compiler_params=pltpu.CompilerParams(needs_layout_passes=False),
        scratch_types=[
            pltpu.VMEM((RG * H,), jnp.float32),
            pltpu.VMEM((2, L), jnp.float32),
        ],
    )
    encode = pl.kernel(
        _encode_body,
        out_type=jax.ShapeDtypeStruct((T_STEPS, ROWS, H), jnp.float32),
        mesh=mesh,
        compiler_params=pltpu.CompilerParams(needs_layout_passes=False),
        scratch_types=[
            pltpu.VMEM((RG * H,), jnp.float32),
            pltpu.VMEM((T_STEPS, RG, H), jnp.float32),
            pltpu.VMEM((RG * H,), jnp.int32),
            pltpu.VMEM((2, L), jnp.float32),
            pltpu.VMEM((NW, 2, L), jnp.float32),
        ],
    )

    def run(flat):
        partials = minmax(flat)
        return encode(flat, partials)

    return run


def kernel(topo_5xwxh):
    flat = topo_5xwxh.reshape(ROWS * H)
    out = _build()(flat)
    return out.reshape(T_STEPS, BANDS, W, H)


# speedup vs baseline: 55.3757x; 1.5769x over previous
import functools

import jax
import jax.numpy as jnp
from jax import lax
from jax.experimental import pallas as pl
from jax.experimental.pallas import tpu as pltpu
from jax.experimental.pallas import tpu_sc as plsc

T_STEPS = 30
BANDS = 5
W = 512
H = 512
ROWS = BANDS * W
NC = 2
NS = 16
L = 16
NW = NC * NS
RPW = ROWS // NW
RG = 2
NG = RPW // RG
VPG = RG * H // L
C1 = 16
NCH = RPW // C1
VPC = C1 * H // L


def _minmax_body(in_hbm, mm_hbm, in0, in1, pmm_v, s0, s1):
    cid = lax.axis_index("c")
    sid = lax.axis_index("s")
    wid = cid * NS + sid
    base = wid * RPW * H
    bufs = (in0, in1)
    sems = (s0, s1)

    def start(k, buf, sem):
        pltpu.async_copy(in_hbm.at[pl.ds(base + k * C1 * H, C1 * H)], buf, sem)

    start(0, in0, s0)
    start(1, in1, s1)

    mn_vec = jnp.full((L,), jnp.inf, jnp.float32)
    mx_vec = -mn_vec
    for k in range(NCH):
        buf, sem = bufs[k % 2], sems[k % 2]
        pltpu.make_async_copy(in_hbm.at[pl.ds(base, C1 * H)], buf, sem).wait()

        def vec(i, c):
            m0, m1 = c
            v = buf[pl.ds(i * L, L)]
            return jnp.minimum(m0, v), jnp.maximum(m1, v)

        mn_vec, mx_vec = lax.fori_loop(0, VPC, vec, (mn_vec, mx_vec))
        if k + 2 < NCH:
            start(k + 2, buf, sem)

    pmm_v[0, :] = mn_vec
    pmm_v[1, :] = mx_vec
    pltpu.sync_copy(pmm_v, mm_hbm.at[wid])


def _encode_body(in_hbm, mm_hbm, out_hbm,
                 in0, in1, blk0, blk1, tid0, tid1, pmm_v, all_v,
                 is0, is1, os0, os1):
    cid = lax.axis_index("c")
    sid = lax.axis_index("s")
    wid = cid * NS + sid

    pltpu.sync_copy(mm_hbm, all_v)
    mn_vec = all_v[0, 0, :]
    mx_vec = all_v[0, 1, :]
    for k in range(1, NW):
        mn_vec = jnp.minimum(mn_vec, all_v[k, 0, :])
        mx_vec = jnp.maximum(mx_vec, all_v[k, 1, :])

    iota = lax.iota(jnp.int32, L)
    row0_idx = jnp.zeros((L,), jnp.int32)
    row1_idx = jnp.full((L,), 1, jnp.int32)
    for s in (8, 4, 2, 1):
        pmm_v[0, :] = mn_vec
        pmm_v[1, :] = mx_vec
        perm = iota ^ s
        mn_vec = jnp.minimum(mn_vec, plsc.load_gather(pmm_v, [row0_idx, perm]))
        mx_vec = jnp.maximum(mx_vec, plsc.load_gather(pmm_v, [row1_idx, perm]))
    mn = mn_vec
    recip = 1.0 / (mx_vec - mn_vec + 1e-8)

    zeros = jnp.zeros((L,), jnp.float32)
    ones = jnp.full((L,), 1.0, jnp.float32)
    ins = (in0, in1)
    blks = (blk0, blk1)
    tids = (tid0, tid1)
    isems = (is0, is1)
    osems = (os0, os1)

    for blk in blks:
        def zero_t(t, _, blk=blk):
            for r in range(RG):
                for j in range(H // L):
                    blk[t, r, pl.ds(j * L, L)] = zeros
            return 0

        lax.fori_loop(0, T_STEPS, zero_t, 0)

    def in_start(g, b):
        row0 = wid * RPW + g * RG
        pltpu.async_copy(in_hbm.at[pl.ds(row0 * H, RG * H)], ins[b], isems[b])

    def in_wait(b):
        pltpu.make_async_copy(in_hbm.at[pl.ds(0, RG * H)], ins[b],
                              isems[b]).wait()

    def out_start(g, b):
        row0 = wid * RPW + g * RG
        pltpu.async_copy(blks[b], out_hbm.at[:, pl.ds(row0, RG), :], osems[b])

    def out_wait(b):
        pltpu.make_async_copy(blks[b], out_hbm.at[:, pl.ds(0, RG), :],
                              osems[b]).wait()

    in_start(0, 0)
    in_start(1, 1)

    def pair(p, _):
        for b in range(2):
            g = p * 2 + b
            blk, tid_v, in_v = blks[b], tids[b], ins[b]
            in_wait(b)

            @pl.when(p >= 1)
            def _():
                out_wait(b)

                def restore_vec(i, _):
                    t = tid_v[pl.ds(i * L, L)]
                    r_vec = jnp.full((L,), (i * L) // H, jnp.int32)
                    j_vec = ((i * L) % H) + iota
                    plsc.store_scatter(blk, [t, r_vec, j_vec], zeros)
                    return 0

                lax.fori_loop(0, VPG, restore_vec, 0)

            def enc_vec(i, _):
                x = in_v[pl.ds(i * L, L)]
                lat = (1.0 - (x - mn) * recip) * (T_STEPS - 1.0)
                t = jnp.clip(lat.astype(jnp.int32), 0, T_STEPS - 1)
                tid_v[pl.ds(i * L, L)] = t
                r_vec = jnp.full((L,), (i * L) // H, jnp.int32)
                j_vec = ((i * L) % H) + iota
                plsc.store_scatter(blk, [t, r_vec, j_vec], ones)
                return 0

            lax.fori_loop(0, VPG, enc_vec, 0)
            out_start(g, b)

            @pl.when(p < NG // 2 - 1)
            def _():
                in_start(g + 2, b)

        return 0

    lax.fori_loop(0, NG // 2, pair, 0)
    out_wait(0)
    out_wait(1)


@functools.cache
def _build():
    mesh = plsc.VectorSubcoreMesh(core_axis_name="c", subcore_axis_name="s")
    minmax = pl.kernel(
        _minmax_body,
        out_type=jax.ShapeDtypeStruct((NW, 2, L), jnp.float32),
        mesh=mesh,
        compiler_params=pltpu.CompilerParams(needs_layout_passes=False),
        scratch_types=[
            pltpu.VMEM((C1 * H,), jnp.float32),
            pltpu.VMEM((C1 * H,), jnp.float32),
            pltpu.VMEM((2, L), jnp.float32),
            pltpu.SemaphoreType.DMA,
            pltpu.SemaphoreType.DMA,
        ],
    )
    encode = pl.kernel(
        _encode_body,
        out_type=jax.ShapeDtypeStruct((T_STEPS, ROWS, H), jnp.float32),
        mesh=mesh,
        compiler_params=pltpu.CompilerParams(needs_layout_passes=False),
        scratch_types=[
            pltpu.VMEM((RG * H,), jnp.float32),
            pltpu.VMEM((RG * H,), jnp.float32),
            pltpu.VMEM((T_STEPS, RG, H), jnp.float32),
            pltpu.VMEM((T_STEPS, RG, H), jnp.float32),
            pltpu.VMEM((RG * H,), jnp.int32),
            pltpu.VMEM((RG * H,), jnp.int32),
            pltpu.VMEM((2, L), jnp.float32),
            pltpu.VMEM((NW, 2, L), jnp.float32),
            pltpu.SemaphoreType.DMA,
            pltpu.SemaphoreType.DMA,
            pltpu.SemaphoreType.DMA,
            pltpu.SemaphoreType.DMA,
        ],
    )

    def run(flat):
        partials = minmax(flat)
        return encode(flat, partials)

    return run


def kernel(topo_5xwxh):
    flat = topo_5xwxh.reshape(ROWS * H)
    out = _build()(flat)
    return out.reshape(T_STEPS, BANDS, W, H)


# speedup vs baseline: 63.3498x; 1.1440x over previous
import functools

import jax
import jax.numpy as jnp
from jax import lax
from jax.experimental import pallas as pl
from jax.experimental.pallas import tpu as pltpu
from jax.experimental.pallas import tpu_sc as plsc

T_STEPS = 30
BANDS = 5
W = 512
H = 512
ROWS = BANDS * W
NC = 2
NS = 16
L = 16
NW = NC * NS
RPW = ROWS // NW
RG = 2
NG = RPW // RG
VPG = RG * H // L


def _minmax_tc_body(x_ref, o_ref):
    x = x_ref[...]
    mn = jnp.min(x)
    mx = jnp.max(x)
    rows = lax.broadcasted_iota(jnp.int32, (8, 128), 0)
    o_ref[...] = jnp.where(rows == 0, mn, mx)


def _encode_body(in_hbm, mm_hbm, out_hbm,
                 in0, in1, blk0, blk1, tid0, tid1, all_v,
                 is0, is1, os0, os1):
    cid = lax.axis_index("c")
    sid = lax.axis_index("s")
    wid = cid * NS + sid

    pltpu.sync_copy(mm_hbm, all_v)
    mn = all_v[0, pl.ds(0, L)]
    mx_vec = all_v[1, pl.ds(0, L)]
    recip = 1.0 / (mx_vec - mn + 1e-8)
    iota = lax.iota(jnp.int32, L)

    zeros = jnp.zeros((L,), jnp.float32)
    ones = jnp.full((L,), 1.0, jnp.float32)
    ins = (in0, in1)
    blks = (blk0, blk1)
    tids = (tid0, tid1)
    isems = (is0, is1)
    osems = (os0, os1)

    for blk in blks:
        def zero_t(t, _, blk=blk):
            for r in range(RG):
                for j in range(H // L):
                    blk[t, r, pl.ds(j * L, L)] = zeros
            return 0

        lax.fori_loop(0, T_STEPS, zero_t, 0)

    def in_start(g, b):
        row0 = wid * RPW + g * RG
        pltpu.async_copy(in_hbm.at[pl.ds(row0 * H, RG * H)], ins[b], isems[b])

    def in_wait(b):
        pltpu.make_async_copy(in_hbm.at[pl.ds(0, RG * H)], ins[b],
                              isems[b]).wait()

    def out_start(g, b):
        row0 = wid * RPW + g * RG
        pltpu.async_copy(blks[b], out_hbm.at[:, pl.ds(row0, RG), :], osems[b])

    def out_wait(b):
        pltpu.make_async_copy(blks[b], out_hbm.at[:, pl.ds(0, RG), :],
                              osems[b]).wait()

    in_start(0, 0)
    in_start(1, 1)

    def pair(p, _):
        for b in range(2):
            g = p * 2 + b
            blk, tid_v, in_v = blks[b], tids[b], ins[b]
            in_wait(b)

            @pl.when(p >= 1)
            def _():
                out_wait(b)

                def restore_vec(i, _):
                    t = tid_v[pl.ds(i * L, L)]
                    r_vec = jnp.full((L,), (i * L) // H, jnp.int32)
                    j_vec = ((i * L) % H) + iota
                    plsc.store_scatter(blk, [t, r_vec, j_vec], zeros)
                    return 0

                lax.fori_loop(0, VPG, restore_vec, 0)

            def enc_vec(i, _):
                x = in_v[pl.ds(i * L, L)]
                lat = (1.0 - (x - mn) * recip) * (T_STEPS - 1.0)
                t = jnp.clip(lat.astype(jnp.int32), 0, T_STEPS - 1)
                tid_v[pl.ds(i * L, L)] = t
                r_vec = jnp.full((L,), (i * L) // H, jnp.int32)
                j_vec = ((i * L) % H) + iota
                plsc.store_scatter(blk, [t, r_vec, j_vec], ones)
                return 0

            lax.fori_loop(0, VPG, enc_vec, 0)
            out_start(g, b)

            @pl.when(p < NG // 2 - 1)
            def _():
                in_start(g + 2, b)

        return 0

    lax.fori_loop(0, NG // 2, pair, 0)
    out_wait(0)
    out_wait(1)


@functools.cache
def _build():
    mesh = plsc.VectorSubcoreMesh(core_axis_name="c", subcore_axis_name="s")
    minmax = pl.pallas_call(
        _minmax_tc_body,
        out_shape=jax.ShapeDtypeStruct((8, 128), jnp.float32),
    )
    encode = pl.kernel(
        _encode_body,
        out_type=jax.ShapeDtypeStruct((T_STEPS, ROWS, H), jnp.float32),
        mesh=mesh,
        compiler_params=pltpu.CompilerParams(needs_layout_passes=False),
        scratch_types=[
            pltpu.VMEM((RG * H,), jnp.float32),
            pltpu.VMEM((RG * H,), jnp.float32),
            pltpu.VMEM((T_STEPS, RG, H), jnp.float32),
            pltpu.VMEM((T_STEPS, RG, H), jnp.float32),
            pltpu.VMEM((RG * H,), jnp.int32),
            pltpu.VMEM((RG * H,), jnp.int32),
            pltpu.VMEM((8, 128), jnp.float32),
            pltpu.SemaphoreType.DMA,
            pltpu.SemaphoreType.DMA,
            pltpu.SemaphoreType.DMA,
            pltpu.SemaphoreType.DMA,
        ],
    )

    def run(flat2d):
        partials = minmax(flat2d)
        return encode(flat2d.reshape(ROWS * H), partials)

    return run


def kernel(topo_5xwxh):
    out = _build()(topo_5xwxh.reshape(ROWS, H))
    return out.reshape(T_STEPS, BANDS, W, H)
